# Initial kernel scaffold; baseline (speedup 1.0000x reference)
import functools

import jax
import jax.numpy as jnp
from jax import lax
from jax.experimental import pallas as pl
from jax.experimental.pallas import tpu as pltpu
from jax.experimental.pallas import tpu_sc as plsc

NC = 2
NS = 16
NW = NC * NS
CHUNK = 128


def _round_up(v, m):
    return (v + m - 1) // m * m


def _deg_body(rows_pad, ept, dst_hbm, zero_hbm, out_hbm, dst_v, ones_v,
              acc_sh):
    c = lax.axis_index("c")
    s = lax.axis_index("s")
    wid = c * NS + s
    rpt = rows_pad // NS
    r0 = s * rpt
    pltpu.sync_copy(zero_hbm.at[pl.ds(r0, rpt)], acc_sh.at[pl.ds(r0, rpt)])
    for i in range(CHUNK // 16):
        ones_v[pl.ds(i * 16, 16)] = jnp.full((16,), 1.0, jnp.float32)
    plsc.subcore_barrier()

    def body(k, carry):
        off = (wid * ept + k) * CHUNK
        pltpu.sync_copy(dst_hbm.at[pl.ds(off, CHUNK)], dst_v)
        pltpu.sync_copy(ones_v, acc_sh.at[dst_v], add=True)
        return carry

    lax.fori_loop(0, ept, body, 0)
    plsc.subcore_barrier()
    pltpu.sync_copy(acc_sh.at[pl.ds(r0, rpt)],
                    out_hbm.at[pl.ds(c * rows_pad + r0, rpt)])


def _spmm_body(rows_pad, ept, d, hs_hbm, src_hbm, dst_hbm, zero_hbm, out_hbm,
               src_v, dst_v, rows_v, acc_sh, sem):
    c = lax.axis_index("c")
    s = lax.axis_index("s")
    wid = c * NS + s
    rpt = rows_pad // NS
    r0 = s * rpt
    pltpu.sync_copy(zero_hbm.at[pl.ds(r0, rpt)], acc_sh.at[pl.ds(r0, rpt)])
    plsc.subcore_barrier()

    def body(k, carry):
        off = (wid * ept + k) * CHUNK
        pltpu.sync_copy(src_hbm.at[pl.ds(off, CHUNK)], src_v)
        pltpu.sync_copy(dst_hbm.at[pl.ds(off, CHUNK)], dst_v)
        pltpu.async_copy(hs_hbm.at[src_v], rows_v, sem).wait()
        pltpu.sync_copy(rows_v, acc_sh.at[dst_v], add=True)
        return carry

    lax.fori_loop(0, ept, body, 0)
    plsc.subcore_barrier()
    pltpu.sync_copy(acc_sh.at[pl.ds(r0, rpt)],
                    out_hbm.at[pl.ds(c * rows_pad + r0, rpt)])


def _make_deg_kernel(rows_pad, ept):
    mesh = plsc.VectorSubcoreMesh(core_axis_name="c", subcore_axis_name="s")
    return pl.kernel(
        functools.partial(_deg_body, rows_pad, ept),
        out_type=jax.ShapeDtypeStruct((NC * rows_pad,), jnp.float32),
        mesh=mesh,
        scratch_types=[
            pltpu.VMEM((CHUNK,), jnp.int32),
            pltpu.VMEM((CHUNK,), jnp.float32),
            pltpu.VMEM_SHARED((rows_pad,), jnp.float32),
        ],
    )


def _make_spmm_kernel(rows_pad, ept, d):
    mesh = plsc.VectorSubcoreMesh(core_axis_name="c", subcore_axis_name="s")
    return pl.kernel(
        functools.partial(_spmm_body, rows_pad, ept, d),
        out_type=jax.ShapeDtypeStruct((NC * rows_pad, d), jnp.float32),
        mesh=mesh,
        scratch_types=[
            pltpu.VMEM((CHUNK,), jnp.int32),
            pltpu.VMEM((CHUNK,), jnp.int32),
            pltpu.VMEM((CHUNK, d), jnp.float32),
            pltpu.VMEM_SHARED((rows_pad, d), jnp.float32),
            pltpu.SemaphoreType.DMA,
        ],
    )


def _tc1_body(x_ref, w_ref, da_ref, db_ref, hs_ref, dis_ref):
    dis = lax.rsqrt(da_ref[...] + db_ref[...] + 1.0)
    t = jnp.dot(x_ref[...], w_ref[...], preferred_element_type=jnp.float32)
    hs_ref[...] = t * dis
    dis_ref[...] = dis


def _tc2_body(aa_ref, ab_ref, hs_ref, dis_ref, b_ref, w_ref, out_ref):
    dis = dis_ref[...]
    pre = (aa_ref[...] + ab_ref[...] + hs_ref[...]) * dis + b_ref[...]
    h = jnp.maximum(pre, 0.0)
    out_ref[...] = dis * jnp.dot(h, w_ref[...],
                                 preferred_element_type=jnp.float32)


def _tc3_body(aa_ref, ab_ref, hs_ref, dis_ref, b_ref, out_ref):
    out_ref[...] = ((aa_ref[...] + ab_ref[...] + hs_ref[...]) * dis_ref[...]
                    + b_ref[...])


def _row_spec(br, d):
    return pl.BlockSpec((br, d), lambda i: (i, 0))


def _full_spec(shape):
    return pl.BlockSpec(shape, lambda i: (0,) * len(shape))


def kernel(x, edge_index, W1, b1, W2, b2):
    n, d = x.shape
    e = edge_index.shape[1]

    src = edge_index[0].astype(jnp.int32)
    dst = edge_index[1].astype(jnp.int32)

    ept = -(-e // (NW * CHUNK))
    e_pad = NW * CHUNK * ept
    rows_pad = _round_up(-(-(n + 1) // NS), 8) * NS

    src_p = jnp.concatenate(
        [src, jnp.zeros((e_pad - e,), jnp.int32)])
    dst_p = jnp.concatenate(
        [dst, jnp.full((e_pad - e,), n, jnp.int32)])

    zero_rows = jnp.zeros((rows_pad, d), jnp.float32)
    zero_deg = jnp.zeros((rows_pad,), jnp.float32)

    deg_out = _make_deg_kernel(rows_pad, ept)(dst_p, zero_deg)
    da = deg_out[0 * rows_pad:0 * rows_pad + n].reshape(n, 1)
    db = deg_out[1 * rows_pad:1 * rows_pad + n].reshape(n, 1)

    br = 2000 if n % 2000 == 0 else n
    grid = (n // br,)

    hs1, dis = pl.pallas_call(
        _tc1_body,
        grid=grid,
        in_specs=[_row_spec(br, d), _full_spec((d, d)),
                  _row_spec(br, 1), _row_spec(br, 1)],
        out_specs=[_row_spec(br, d), _row_spec(br, 1)],
        out_shape=[jax.ShapeDtypeStruct((n, d), jnp.float32),
                   jax.ShapeDtypeStruct((n, 1), jnp.float32)],
    )(x, W1, da, db)

    spmm = _make_spmm_kernel(rows_pad, ept, d)

    agg1 = spmm(hs1, src_p, dst_p, zero_rows)
    a1a = agg1[0 * rows_pad:0 * rows_pad + n]
    a1b = agg1[1 * rows_pad:1 * rows_pad + n]

    hs2 = pl.pallas_call(
        _tc2_body,
        grid=grid,
        in_specs=[_row_spec(br, d), _row_spec(br, d), _row_spec(br, d),
                  _row_spec(br, 1), _full_spec((1, d)), _full_spec((d, d))],
        out_specs=_row_spec(br, d),
        out_shape=jax.ShapeDtypeStruct((n, d), jnp.float32),
    )(a1a, a1b, hs1, dis, b1.reshape(1, d), W2)

    agg2 = spmm(hs2, src_p, dst_p, zero_rows)
    a2a = agg2[0 * rows_pad:0 * rows_pad + n]
    a2b = agg2[1 * rows_pad:1 * rows_pad + n]

    out = pl.pallas_call(
        _tc3_body,
        grid=grid,
        in_specs=[_row_spec(br, d), _row_spec(br, d), _row_spec(br, d),
                  _row_spec(br, 1), _full_spec((1, d))],
        out_specs=_row_spec(br, d),
        out_shape=jax.ShapeDtypeStruct((n, d), jnp.float32),
    )(a2a, a2b, hs2, dis, b2.reshape(1, d))

    return out


# speedup vs baseline: 11.4584x; 11.4584x over previous
import functools

import jax
import jax.numpy as jnp
from jax import lax
from jax.experimental import pallas as pl
from jax.experimental.pallas import tpu as pltpu
from jax.experimental.pallas import tpu_sc as plsc

NC = 2
NS = 16
NW = NC * NS
CHUNK = 128


def _round_up(v, m):
    return (v + m - 1) // m * m


def _deg_body(rows_pad, ept, dst_hbm, out_hbm, dst_v, ones_v, stage_v,
              acc_sh):
    c = lax.axis_index("c")
    s = lax.axis_index("s")
    wid = c * NS + s
    rpt = rows_pad // NS
    r0 = s * rpt
    zero16 = jnp.zeros((16,), jnp.float32)
    for i in range(CHUNK // 16):
        ones_v[pl.ds(i * 16, 16)] = jnp.full((16,), 1.0, jnp.float32)

    def zbody(k, carry):
        stage_v[pl.ds(k * 16, 16)] = zero16
        return carry

    lax.fori_loop(0, rpt // 16, zbody, 0)
    pltpu.sync_copy(stage_v, acc_sh.at[pl.ds(r0, rpt)])
    plsc.subcore_barrier()

    def body(k, carry):
        off = (wid * ept + k) * CHUNK
        pltpu.sync_copy(dst_hbm.at[pl.ds(off, CHUNK)], dst_v)
        pltpu.sync_copy(ones_v, acc_sh.at[dst_v], add=True)
        return carry

    lax.fori_loop(0, ept, body, 0)
    plsc.subcore_barrier()
    pltpu.sync_copy(acc_sh.at[pl.ds(r0, rpt)], stage_v)
    pltpu.sync_copy(stage_v, out_hbm.at[pl.ds(c * rows_pad + r0, rpt)])


def _spmm_body(rows_pad, ept, d, hs_hbm, src_hbm, dst_hbm, out_hbm,
               src_v, dst_v, rows_v, acc_sh, sem):
    c = lax.axis_index("c")
    s = lax.axis_index("s")
    wid = c * NS + s
    rpt = rows_pad // NS
    r0 = s * rpt
    nstage = rpt // CHUNK
    zero16 = jnp.zeros((16,), jnp.float32)

    def zbody(k, carry):
        for j in range(d // 16):
            rows_v[k, pl.ds(j * 16, 16)] = zero16
        return carry

    lax.fori_loop(0, CHUNK, zbody, 0)
    for i in range(nstage):
        pltpu.sync_copy(rows_v, acc_sh.at[pl.ds(r0 + i * CHUNK, CHUNK)])
    plsc.subcore_barrier()

    def body(k, carry):
        off = (wid * ept + k) * CHUNK
        pltpu.sync_copy(src_hbm.at[pl.ds(off, CHUNK)], src_v)
        pltpu.sync_copy(dst_hbm.at[pl.ds(off, CHUNK)], dst_v)
        pltpu.async_copy(hs_hbm.at[src_v], rows_v, sem).wait()
        pltpu.sync_copy(rows_v, acc_sh.at[dst_v], add=True)
        return carry

    lax.fori_loop(0, ept, body, 0)
    plsc.subcore_barrier()
    for i in range(nstage):
        pltpu.sync_copy(acc_sh.at[pl.ds(r0 + i * CHUNK, CHUNK)], rows_v)
        pltpu.sync_copy(rows_v,
                        out_hbm.at[pl.ds(c * rows_pad + r0 + i * CHUNK,
                                         CHUNK)])


def _make_deg_kernel(rows_pad, ept):
    mesh = plsc.VectorSubcoreMesh(core_axis_name="c", subcore_axis_name="s")
    return pl.kernel(
        functools.partial(_deg_body, rows_pad, ept),
        out_type=jax.ShapeDtypeStruct((NC * rows_pad,), jnp.float32),
        mesh=mesh,
        scratch_types=[
            pltpu.VMEM((CHUNK,), jnp.int32),
            pltpu.VMEM((CHUNK,), jnp.float32),
            pltpu.VMEM((rows_pad // NS,), jnp.float32),
            pltpu.VMEM_SHARED((rows_pad,), jnp.float32),
        ],
    )


def _make_spmm_kernel(rows_pad, ept, d):
    mesh = plsc.VectorSubcoreMesh(core_axis_name="c", subcore_axis_name="s")
    return pl.kernel(
        functools.partial(_spmm_body, rows_pad, ept, d),
        out_type=jax.ShapeDtypeStruct((NC * rows_pad, d), jnp.float32),
        mesh=mesh,
        scratch_types=[
            pltpu.VMEM((CHUNK,), jnp.int32),
            pltpu.VMEM((CHUNK,), jnp.int32),
            pltpu.VMEM((CHUNK, d), jnp.float32),
            pltpu.VMEM_SHARED((rows_pad, d), jnp.float32),
            pltpu.SemaphoreType.DMA,
        ],
    )


def _tc1_body(x_ref, w_ref, da_ref, db_ref, hs_ref, dis_ref):
    dis = lax.rsqrt(da_ref[...] + db_ref[...] + 1.0)
    t = jnp.dot(x_ref[...], w_ref[...], preferred_element_type=jnp.float32)
    hs_ref[...] = t * dis
    dis_ref[...] = dis


def _tc2_body(aa_ref, ab_ref, hs_ref, dis_ref, b_ref, w_ref, out_ref):
    dis = dis_ref[...]
    pre = (aa_ref[...] + ab_ref[...] + hs_ref[...]) * dis + b_ref[...]
    h = jnp.maximum(pre, 0.0)
    out_ref[...] = dis * jnp.dot(h, w_ref[...],
                                 preferred_element_type=jnp.float32)


def _tc3_body(aa_ref, ab_ref, hs_ref, dis_ref, b_ref, out_ref):
    out_ref[...] = ((aa_ref[...] + ab_ref[...] + hs_ref[...]) * dis_ref[...]
                    + b_ref[...])


def _row_spec(br, d):
    return pl.BlockSpec((br, d), lambda i: (i, 0))


def _full_spec(shape):
    return pl.BlockSpec(shape, lambda i: (0,) * len(shape))


def kernel(x, edge_index, W1, b1, W2, b2):
    n, d = x.shape
    e = edge_index.shape[1]

    src = edge_index[0].astype(jnp.int32)
    dst = edge_index[1].astype(jnp.int32)

    ept = -(-e // (NW * CHUNK))
    e_pad = NW * CHUNK * ept
    rows_pad = _round_up(n + 1, NS * CHUNK)

    src_p = jnp.concatenate(
        [src, jnp.zeros((e_pad - e,), jnp.int32)])
    dst_p = jnp.concatenate(
        [dst, jnp.full((e_pad - e,), n, jnp.int32)])

    deg_out = _make_deg_kernel(rows_pad, ept)(dst_p)
    da = deg_out[0 * rows_pad:0 * rows_pad + n].reshape(n, 1)
    db = deg_out[1 * rows_pad:1 * rows_pad + n].reshape(n, 1)

    br = 2000 if n % 2000 == 0 else n
    grid = (n // br,)

    hs1, dis = pl.pallas_call(
        _tc1_body,
        grid=grid,
        in_specs=[_row_spec(br, d), _full_spec((d, d)),
                  _row_spec(br, 1), _row_spec(br, 1)],
        out_specs=[_row_spec(br, d), _row_spec(br, 1)],
        out_shape=[jax.ShapeDtypeStruct((n, d), jnp.float32),
                   jax.ShapeDtypeStruct((n, 1), jnp.float32)],
    )(x, W1, da, db)

    spmm = _make_spmm_kernel(rows_pad, ept, d)

    agg1 = spmm(hs1, src_p, dst_p)
    a1a = agg1[0 * rows_pad:0 * rows_pad + n]
    a1b = agg1[1 * rows_pad:1 * rows_pad + n]

    hs2 = pl.pallas_call(
        _tc2_body,
        grid=grid,
        in_specs=[_row_spec(br, d), _row_spec(br, d), _row_spec(br, d),
                  _row_spec(br, 1), _full_spec((1, d)), _full_spec((d, d))],
        out_specs=_row_spec(br, d),
        out_shape=jax.ShapeDtypeStruct((n, d), jnp.float32),
    )(a1a, a1b, hs1, dis, b1.reshape(1, d), W2)

    agg2 = spmm(hs2, src_p, dst_p)
    a2a = agg2[0 * rows_pad:0 * rows_pad + n]
    a2b = agg2[1 * rows_pad:1 * rows_pad + n]

    out = pl.pallas_call(
        _tc3_body,
        grid=grid,
        in_specs=[_row_spec(br, d), _row_spec(br, d), _row_spec(br, d),
                  _row_spec(br, 1), _full_spec((1, d))],
        out_specs=_row_spec(br, d),
        out_shape=jax.ShapeDtypeStruct((n, d), jnp.float32),
    )(a2a, a2b, hs2, dis, b2.reshape(1, d))

    return out
